# baseline (speedup 1.0000x reference)
import functools

import jax
import jax.numpy as jnp
from jax import lax
from jax.experimental import pallas as pl
from jax.experimental.pallas import tpu as pltpu
from jax.experimental.pallas import tpu_sc as plsc

N = 144242
D = 128
E = 600000

R = 9216
NB = 16
N_PAD = R * NB
ROWS_PER_TILE = R // 16

CHUNK = 512
E_PAD = 606208
NCHUNK = E_PAD // 16 // CHUNK
K = 4096
F = 128

TC_BLK = 512
TC_GRID = N_PAD // TC_BLK
TC_LAST = (N + TC_BLK - 1) // TC_BLK - 1


def _tc_body(emb_ref, w1_ref, w2_ref, b1_ref, b2_ref, y_ref, base_ref):
    e = emb_ref[...]
    a = lax.dot_general(e, w1_ref[...], (((1,), (1,)), ((), ())),
                        precision=lax.Precision.HIGHEST,
                        preferred_element_type=jnp.float32)
    b = lax.dot_general(e * e, w2_ref[...], (((1,), (1,)), ((), ())),
                        precision=lax.Precision.HIGHEST,
                        preferred_element_type=jnp.float32)
    y_ref[...] = a + b
    base_ref[...] = a + (b1_ref[...] + b2_ref[...])


def _tc_dense(embed, W1, W2, b1, b2):
    return pl.pallas_call(
        _tc_body,
        grid=(TC_GRID,),
        in_specs=[
            pl.BlockSpec((TC_BLK, D), lambda i: (jnp.minimum(i, TC_LAST), 0)),
            pl.BlockSpec((D, D), lambda i: (0, 0)),
            pl.BlockSpec((D, D), lambda i: (0, 0)),
            pl.BlockSpec((1, D), lambda i: (0, 0)),
            pl.BlockSpec((1, D), lambda i: (0, 0)),
        ],
        out_specs=[
            pl.BlockSpec((TC_BLK, D), lambda i: (i, 0)),
            pl.BlockSpec((TC_BLK, D), lambda i: (i, 0)),
        ],
        out_shape=[
            jax.ShapeDtypeStruct((N_PAD, D), jnp.float32),
            jax.ShapeDtypeStruct((N_PAD, D), jnp.float32),
        ],
    )(embed, W1, W2, b1, b2)


def _sc_spmm_body(ed_h, y_h, base_h, out_h,
                  eb_a, eb_b, st_r, st_c, st_w,
                  ridx, cidx, rowsv, ridx2, cidx2, rowsv2,
                  acc, sem_a, sem_b, sem_ga, sem_gb):
    cid = lax.axis_index("c")
    sid = lax.axis_index("s")
    zero16i = jnp.zeros((16,), jnp.int32)
    zero16f = jnp.zeros((16,), jnp.float32)

    def init_body(t, _):
        st_r[pl.ds(16 * t, 16)] = zero16i
        st_c[pl.ds(16 * t, 16)] = zero16i
        return 0
    lax.fori_loop(0, (K + 16) // 16, init_body, 0)

    def init_w(t, _):
        st_w[pl.ds(16 * t, 16)] = zero16f
        return 0
    lax.fori_loop(0, (K + 2 * F) // 16, init_w, 0)

    def flush(cnt):
        def ztail(t, _):
            st_w[pl.ds(cnt + 16 * t, 16)] = zero16f
            return 0
        lax.fori_loop(0, 2 * F // 16, ztail, 0)
        npair = (cnt + 2 * F - 1) // (2 * F)
        smax = K // F - 1

        def prep(s, rx, cx):
            def cp(t, _):
                cx[pl.ds(16 * t, 16)] = st_c[pl.ds(s * F + 16 * t, 16)]
                rx[pl.ds(16 * t, 16)] = st_r[pl.ds(s * F + 16 * t, 16)]
                return 0
            lax.fori_loop(0, F // 16, cp, 0)

        def scale(rv, off):
            def body(i, _):
                wsp = plsc.load_gather(
                    st_w, [jnp.full((16,), off + i, jnp.int32)])
                for dd in range(8):
                    sl = pl.ds(16 * dd, 16)
                    rv[i, sl] = rv[i, sl] * wsp
                return 0
            lax.fori_loop(0, F, body, 0, unroll=4)

        prep(0, ridx, cidx)
        pltpu.async_copy(y_h.at[cidx], rowsv, sem_ga)

        def pb(p, _):
            s0 = 2 * p
            prep(s0 + 1, ridx2, cidx2)
            pltpu.async_copy(y_h.at[cidx2], rowsv2, sem_gb)
            pltpu.make_async_copy(y_h.at[cidx], rowsv, sem_ga).wait()
            scale(rowsv, s0 * F)
            pltpu.sync_copy(rowsv, acc.at[ridx], add=True)
            prep(jnp.minimum(s0 + 2, smax), ridx, cidx)
            pltpu.async_copy(y_h.at[cidx], rowsv, sem_ga)
            pltpu.make_async_copy(y_h.at[cidx2], rowsv2, sem_gb).wait()
            scale(rowsv2, (s0 + 1) * F)
            pltpu.sync_copy(rowsv2, acc.at[ridx2], add=True)
            return 0
        lax.fori_loop(0, npair, pb, 0)
        pltpu.make_async_copy(y_h.at[cidx], rowsv, sem_ga).wait()
        return 0

    def block_body(ib, _):
        b = 2 * ib + cid
        _do_block(b * R)
        return 0

    cb = sid * NCHUNK

    def _do_block(lo):
        pltpu.sync_copy(base_h.at[pl.ds(lo + sid * ROWS_PER_TILE, ROWS_PER_TILE)],
                        acc.at[pl.ds(sid * ROWS_PER_TILE, ROWS_PER_TILE)])
        plsc.subcore_barrier()

        def process(eb, cnt):
            def j_body(j, cnt):
                sl = pl.ds(16 * j, 16)
                r16 = eb[0, sl]
                m = (r16 >= lo) & (r16 < lo + R)
                cs = plsc.cumsum(m.astype(jnp.int32))
                tgt = cnt + cs - 1
                plsc.store_scatter(st_r, [tgt], r16 - lo, mask=m)
                plsc.store_scatter(st_c, [tgt], eb[1, sl], mask=m)
                plsc.store_scatter(st_w, [tgt],
                                   plsc.bitcast(eb[2, sl], jnp.float32), mask=m)
                return cnt + cs[15]
            cnt = lax.fori_loop(0, CHUNK // 16, j_body, cnt)
            return lax.cond(cnt > K - CHUNK, flush, lambda c: c, cnt)

        pltpu.async_copy(ed_h.at[cb], eb_a, sem_a)

        def pair_body(p, cnt):
            ci = 2 * p
            pltpu.async_copy(ed_h.at[cb + ci + 1], eb_b, sem_b)
            pltpu.make_async_copy(ed_h.at[cb + ci], eb_a, sem_a).wait()
            cnt = process(eb_a, cnt)
            nxt = jnp.minimum(ci + 2, NCHUNK - 2)
            pltpu.async_copy(ed_h.at[cb + nxt], eb_a, sem_a)
            pltpu.make_async_copy(ed_h.at[cb + ci + 1], eb_b, sem_b).wait()
            return process(eb_b, cnt)

        cnt = lax.fori_loop(0, NCHUNK // 2, pair_body, 0)
        pltpu.make_async_copy(ed_h.at[cb + NCHUNK - 2], eb_a, sem_a).wait()
        lax.cond(cnt > 0, flush, lambda c: c, cnt)
        plsc.subcore_barrier()

        pltpu.sync_copy(acc.at[pl.ds(sid * ROWS_PER_TILE, ROWS_PER_TILE)],
                        out_h.at[pl.ds(lo + sid * ROWS_PER_TILE, ROWS_PER_TILE)])
        return 0

    lax.fori_loop(0, NB // 2, block_body, 0)


_sc_spmm = functools.partial(
    pl.kernel,
    out_type=jax.ShapeDtypeStruct((N_PAD, D), jnp.float32),
    mesh=plsc.VectorSubcoreMesh(core_axis_name="c", subcore_axis_name="s"),
    compiler_params=pltpu.CompilerParams(needs_layout_passes=False),
    scratch_types=[
        pltpu.VMEM((3, CHUNK), jnp.int32),
        pltpu.VMEM((3, CHUNK), jnp.int32),
        pltpu.VMEM((K + 16,), jnp.int32),
        pltpu.VMEM((K + 16,), jnp.int32),
        pltpu.VMEM((K + 2 * F,), jnp.float32),
        pltpu.VMEM((F,), jnp.int32),
        pltpu.VMEM((F,), jnp.int32),
        pltpu.VMEM((F, D), jnp.float32),
        pltpu.VMEM((F,), jnp.int32),
        pltpu.VMEM((F,), jnp.int32),
        pltpu.VMEM((F, D), jnp.float32),
        pltpu.VMEM_SHARED((R, D), jnp.float32),
        pltpu.SemaphoreType.DMA,
        pltpu.SemaphoreType.DMA,
        pltpu.SemaphoreType.DMA,
        pltpu.SemaphoreType.DMA,
    ],
)(_sc_spmm_body)


def kernel(embed, edge_index, edge_weight, W1, b1, W2, b2):
    pad = E_PAD - E
    rows_p = jnp.concatenate([edge_index[0], jnp.zeros((pad,), jnp.int32)])
    cols_p = jnp.concatenate([edge_index[1], jnp.zeros((pad,), jnp.int32)])
    w_p = jnp.concatenate([
        lax.bitcast_convert_type(edge_weight, jnp.int32),
        jnp.zeros((pad,), jnp.int32)])
    ed = jnp.stack([rows_p.reshape(-1, CHUNK),
                    cols_p.reshape(-1, CHUNK),
                    w_p.reshape(-1, CHUNK)], axis=1)
    y, base = _tc_dense(embed, W1, W2,
                        b1.reshape(1, D), b2.reshape(1, D))
    out_pad = _sc_spmm(ed, y, base)
    return out_pad[:N, :]


# speedup vs baseline: 1.7023x; 1.7023x over previous
import functools

import jax
import jax.numpy as jnp
from jax import lax
from jax.experimental import pallas as pl
from jax.experimental.pallas import tpu as pltpu
from jax.experimental.pallas import tpu_sc as plsc

N = 144242
D = 128
E = 600000

R = 9216
NB = 16
N_PAD = R * NB
ROWS_PER_TILE = R // 16

CHUNK = 512
E_PAD = 606208
NCHUNK = E_PAD // 16 // CHUNK
K = 4096
F = 128

TC_BLK = 512
TC_GRID = N_PAD // TC_BLK
TC_LAST = (N + TC_BLK - 1) // TC_BLK - 1


def _tc_body(emb_ref, w1_ref, w2_ref, b1_ref, b2_ref, y_ref, base_ref):
    e = emb_ref[...]
    a = lax.dot_general(e, w1_ref[...], (((1,), (1,)), ((), ())),
                        precision=lax.Precision.HIGHEST,
                        preferred_element_type=jnp.float32)
    b = lax.dot_general(e * e, w2_ref[...], (((1,), (1,)), ((), ())),
                        precision=lax.Precision.HIGHEST,
                        preferred_element_type=jnp.float32)
    y_ref[...] = a + b
    base_ref[...] = a + (b1_ref[...] + b2_ref[...])


def _tc_dense(embed, W1, W2, b1, b2):
    return pl.pallas_call(
        _tc_body,
        grid=(TC_GRID,),
        in_specs=[
            pl.BlockSpec((TC_BLK, D), lambda i: (jnp.minimum(i, TC_LAST), 0)),
            pl.BlockSpec((D, D), lambda i: (0, 0)),
            pl.BlockSpec((D, D), lambda i: (0, 0)),
            pl.BlockSpec((1, D), lambda i: (0, 0)),
            pl.BlockSpec((1, D), lambda i: (0, 0)),
        ],
        out_specs=[
            pl.BlockSpec((TC_BLK, D), lambda i: (i, 0)),
            pl.BlockSpec((TC_BLK, D), lambda i: (i, 0)),
        ],
        out_shape=[
            jax.ShapeDtypeStruct((N_PAD, D), jnp.float32),
            jax.ShapeDtypeStruct((N_PAD, D), jnp.float32),
        ],
    )(embed, W1, W2, b1, b2)


def _sc_spmm_body(ed_h, y_h, base_h, out_h,
                  eb_a, eb_b, st_r, st_c, st_w,
                  ridx, cidx, rowsv, ridx2, cidx2, rowsv2,
                  acc, sem_a, sem_b, sem_ga, sem_gb):
    cid = lax.axis_index("c")
    sid = lax.axis_index("s")
    zero16i = jnp.zeros((16,), jnp.int32)
    zero16f = jnp.zeros((16,), jnp.float32)

    def init_body(t, _):
        st_r[pl.ds(16 * t, 16)] = zero16i
        st_c[pl.ds(16 * t, 16)] = zero16i
        return 0
    lax.fori_loop(0, (K + 16) // 16, init_body, 0)

    def init_w(t, _):
        st_w[pl.ds(16 * t, 16)] = zero16f
        return 0
    lax.fori_loop(0, (K + 2 * F) // 16, init_w, 0)

    def flush(cnt):
        def ztail(t, _):
            st_w[pl.ds(cnt + 16 * t, 16)] = zero16f
            return 0
        lax.fori_loop(0, F // 16, ztail, 0)
        nsub = (cnt + F - 1) // F

        def sub(s, _):
            off = s * F

            def cp(t, _):
                cidx[pl.ds(16 * t, 16)] = st_c[pl.ds(off + 16 * t, 16)]
                ridx[pl.ds(16 * t, 16)] = st_r[pl.ds(off + 16 * t, 16)]
                return 0
            lax.fori_loop(0, F // 16, cp, 0)
            pltpu.sync_copy(y_h.at[cidx], rowsv)

            def scale(i, _):
                wsp = plsc.load_gather(
                    st_w, [jnp.full((16,), off + i, jnp.int32)])
                for dd in range(8):
                    sl = pl.ds(16 * dd, 16)
                    rowsv[i, sl] = rowsv[i, sl] * wsp
                return 0
            lax.fori_loop(0, F, scale, 0, unroll=4)
            pltpu.sync_copy(rowsv, acc.at[ridx], add=True)
            return 0
        lax.fori_loop(0, nsub, sub, 0)
        return 0

    def block_body(ib, _):
        b = 2 * ib + cid
        _do_block(b * R)
        return 0

    cb = sid * NCHUNK

    def _do_block(lo):
        pltpu.sync_copy(base_h.at[pl.ds(lo + sid * ROWS_PER_TILE, ROWS_PER_TILE)],
                        acc.at[pl.ds(sid * ROWS_PER_TILE, ROWS_PER_TILE)])
        plsc.subcore_barrier()

        def process(eb, cnt):
            def j_body(j, cnt):
                sl = pl.ds(16 * j, 16)
                r16 = eb[0, sl]
                m = (r16 >= lo) & (r16 < lo + R)
                cs = plsc.cumsum(m.astype(jnp.int32))
                tgt = cnt + cs - 1
                plsc.store_scatter(st_r, [tgt], r16 - lo, mask=m)
                plsc.store_scatter(st_c, [tgt], eb[1, sl], mask=m)
                plsc.store_scatter(st_w, [tgt],
                                   plsc.bitcast(eb[2, sl], jnp.float32), mask=m)
                return cnt + cs[15]
            cnt = lax.fori_loop(0, CHUNK // 16, j_body, cnt)
            return lax.cond(cnt > K - CHUNK, flush, lambda c: c, cnt)

        pltpu.async_copy(ed_h.at[cb], eb_a, sem_a)

        def pair_body(p, cnt):
            ci = 2 * p
            pltpu.async_copy(ed_h.at[cb + ci + 1], eb_b, sem_b)
            pltpu.make_async_copy(ed_h.at[cb + ci], eb_a, sem_a).wait()
            cnt = process(eb_a, cnt)
            nxt = jnp.minimum(ci + 2, NCHUNK - 2)
            pltpu.async_copy(ed_h.at[cb + nxt], eb_a, sem_a)
            pltpu.make_async_copy(ed_h.at[cb + ci + 1], eb_b, sem_b).wait()
            return process(eb_b, cnt)

        cnt = lax.fori_loop(0, NCHUNK // 2, pair_body, 0)
        pltpu.make_async_copy(ed_h.at[cb + NCHUNK - 2], eb_a, sem_a).wait()
        lax.cond(cnt > 0, flush, lambda c: c, cnt)
        plsc.subcore_barrier()

        pltpu.sync_copy(acc.at[pl.ds(sid * ROWS_PER_TILE, ROWS_PER_TILE)],
                        out_h.at[pl.ds(lo + sid * ROWS_PER_TILE, ROWS_PER_TILE)])
        return 0

    lax.fori_loop(0, NB // 2, block_body, 0)


_sc_spmm = functools.partial(
    pl.kernel,
    out_type=jax.ShapeDtypeStruct((N_PAD, D), jnp.float32),
    mesh=plsc.VectorSubcoreMesh(core_axis_name="c", subcore_axis_name="s"),
    compiler_params=pltpu.CompilerParams(needs_layout_passes=False),
    scratch_types=[
        pltpu.VMEM((3, CHUNK), jnp.int32),
        pltpu.VMEM((3, CHUNK), jnp.int32),
        pltpu.VMEM((K + 16,), jnp.int32),
        pltpu.VMEM((K + 16,), jnp.int32),
        pltpu.VMEM((K + 2 * F,), jnp.float32),
        pltpu.VMEM((F,), jnp.int32),
        pltpu.VMEM((F,), jnp.int32),
        pltpu.VMEM((F, D), jnp.float32),
        pltpu.VMEM((F,), jnp.int32),
        pltpu.VMEM((F,), jnp.int32),
        pltpu.VMEM((F, D), jnp.float32),
        pltpu.VMEM_SHARED((R, D), jnp.float32),
        pltpu.SemaphoreType.DMA,
        pltpu.SemaphoreType.DMA,
        pltpu.SemaphoreType.DMA,
        pltpu.SemaphoreType.DMA,
    ],
)(_sc_spmm_body)


def kernel(embed, edge_index, edge_weight, W1, b1, W2, b2):
    pad = E_PAD - E
    rows_p = jnp.concatenate([edge_index[0], jnp.zeros((pad,), jnp.int32)])
    cols_p = jnp.concatenate([edge_index[1], jnp.zeros((pad,), jnp.int32)])
    w_p = jnp.concatenate([
        lax.bitcast_convert_type(edge_weight, jnp.int32),
        jnp.zeros((pad,), jnp.int32)])
    ed = jnp.stack([rows_p.reshape(-1, CHUNK),
                    cols_p.reshape(-1, CHUNK),
                    w_p.reshape(-1, CHUNK)], axis=1)
    y, base = _tc_dense(embed, W1, W2,
                        b1.reshape(1, D), b2.reshape(1, D))
    out_pad = _sc_spmm(ed, y, base)
    return out_pad[:N, :]
